# baseline (speedup 1.0000x reference)
import functools

import jax
import jax.numpy as jnp
from jax import lax
from jax.experimental import pallas as pl
from jax.experimental.pallas import tpu as pltpu
from jax.experimental.pallas import tpu_sc as plsc

BATCH = 16384
NC, NS, L = 2, 16, 16
NW = NC * NS
B_PER_W = BATCH // NW
CHUNK = 128
NCHUNK = B_PER_W // CHUNK
D = 64
IDXW = 128
BLKR = 8192


def _score_kernel(w_ref, x_ref, o_ref):
    o_ref[...] = lax.dot_general(
        w_ref[...], x_ref[...],
        dimension_numbers=(((1,), (1,)), ((), ())),
        preferred_element_type=jnp.float32)[None]


def _scores(table, w_half):
    rows = table.shape[0]
    grid = (rows + BLKR - 1) // BLKR
    out = pl.pallas_call(
        _score_kernel,
        grid=(grid,),
        in_specs=[
            pl.BlockSpec((1, D), lambda i: (0, 0)),
            pl.BlockSpec((BLKR, D), lambda i: (i, 0)),
        ],
        out_specs=pl.BlockSpec((1, 1, BLKR), lambda i: (i, 0, 0)),
        out_shape=jax.ShapeDtypeStruct((grid, 1, BLKR), jnp.float32),
    )(w_half, table)
    return out.reshape(grid * BLKR // CHUNK, CHUNK)


def _lookup_kernel(urow_hbm, ucol_hbm, irow_hbm, icol_hbm,
                   us_hbm, is_hbm, b_hbm, out_hbm,
                   urow_v, ucol_v, irow_v, icol_v,
                   usc_v, isc_v, b_v, logit_v, usem, isem):
    wid = lax.axis_index("s") * NC + lax.axis_index("c")
    base = wid * NCHUNK

    pltpu.sync_copy(urow_hbm.at[pl.ds(base, NCHUNK)], urow_v)
    pltpu.sync_copy(ucol_hbm.at[pl.ds(base, NCHUNK)], ucol_v)
    pltpu.sync_copy(irow_hbm.at[pl.ds(base, NCHUNK)], irow_v)
    pltpu.sync_copy(icol_hbm.at[pl.ds(base, NCHUNK)], icol_v)
    pltpu.sync_copy(b_hbm, b_v)

    bias = b_v[pl.ds(0, L)]
    rows0 = lax.iota(jnp.int32, L)

    for j in range(NCHUNK):
        cu = pltpu.async_copy(us_hbm.at[urow_v.at[j]], usc_v, usem)
        ci = pltpu.async_copy(is_hbm.at[irow_v.at[j]], isc_v, isem)
        cu.wait()
        ci.wait()

        def group_body(g, _, j=j):
            rows = rows0 + g * L
            uv = plsc.load_gather(usc_v, [rows, ucol_v[j, pl.ds(g * L, L)]])
            iv = plsc.load_gather(isc_v, [rows, icol_v[j, pl.ds(g * L, L)]])
            x = uv + iv + bias
            logit_v[pl.ds(j * CHUNK + g * L, L)] = 1.0 / (1.0 + jnp.exp(-x))
            return 0

        lax.fori_loop(0, CHUNK // L, group_body, 0)

    pltpu.sync_copy(logit_v, out_hbm.at[pl.ds(wid * B_PER_W, B_PER_W)])


@jax.jit
def kernel(batch_user_id, batch_item_id, user_table, item_table, W, b):
    uid = batch_user_id.astype(jnp.int32)
    iid = batch_item_id.astype(jnp.int32)
    urow = (uid >> 7).reshape(BATCH // IDXW, IDXW)
    ucol = (uid & 127).reshape(BATCH // IDXW, IDXW)
    irow = (iid >> 7).reshape(BATCH // IDXW, IDXW)
    icol = (iid & 127).reshape(BATCH // IDXW, IDXW)
    b16 = jnp.broadcast_to(b, (L,))

    us2 = _scores(user_table, W[:D].reshape(1, D))
    is2 = _scores(item_table, W[D:].reshape(1, D))

    run = functools.partial(
        pl.kernel,
        out_type=jax.ShapeDtypeStruct((BATCH,), jnp.float32),
        mesh=plsc.VectorSubcoreMesh(core_axis_name="c", subcore_axis_name="s"),
        compiler_params=pltpu.CompilerParams(needs_layout_passes=False),
        scratch_types=[
            pltpu.VMEM((NCHUNK, IDXW), jnp.int32),
            pltpu.VMEM((NCHUNK, IDXW), jnp.int32),
            pltpu.VMEM((NCHUNK, IDXW), jnp.int32),
            pltpu.VMEM((NCHUNK, IDXW), jnp.int32),
            pltpu.VMEM((CHUNK, CHUNK), jnp.float32),
            pltpu.VMEM((CHUNK, CHUNK), jnp.float32),
            pltpu.VMEM((L,), jnp.float32),
            pltpu.VMEM((B_PER_W,), jnp.float32),
            pltpu.SemaphoreType.DMA,
            pltpu.SemaphoreType.DMA,
        ],
    )(_lookup_kernel)
    out = run(urow, ucol, irow, icol, us2, is2, b16)
    return out.reshape(BATCH, 1)


# speedup vs baseline: 1.0417x; 1.0417x over previous
import functools

import jax
import jax.numpy as jnp
from jax import lax
from jax.experimental import pallas as pl
from jax.experimental.pallas import tpu as pltpu
from jax.experimental.pallas import tpu_sc as plsc

BATCH = 16384
NC, NS, L = 2, 16, 16
NW = NC * NS
B_PER_W = BATCH // NW
CHUNK = 128
NCHUNK = B_PER_W // CHUNK
D = 64
IDXW = 128
BLKR = 32768


def _score_kernel(w_ref, x_ref, o_ref):
    o_ref[...] = lax.dot_general(
        w_ref[...], x_ref[...],
        dimension_numbers=(((1,), (1,)), ((), ())),
        preferred_element_type=jnp.float32)[None]


def _scores(table, w_half):
    rows = table.shape[0]
    grid = (rows + BLKR - 1) // BLKR
    out = pl.pallas_call(
        _score_kernel,
        grid=(grid,),
        in_specs=[
            pl.BlockSpec((1, D), lambda i: (0, 0)),
            pl.BlockSpec((BLKR, D), lambda i: (i, 0)),
        ],
        out_specs=pl.BlockSpec((1, 1, BLKR), lambda i: (i, 0, 0)),
        out_shape=jax.ShapeDtypeStruct((grid, 1, BLKR), jnp.float32),
    )(w_half, table)
    return out.reshape(grid * BLKR // CHUNK, CHUNK)


def _lookup_kernel(urow_hbm, ucol_hbm, irow_hbm, icol_hbm,
                   us_hbm, is_hbm, b_hbm, out_hbm,
                   urow_v, ucol_v, irow_v, icol_v,
                   usc_v, isc_v, b_v, logit_v, usem, isem):
    wid = lax.axis_index("s") * NC + lax.axis_index("c")
    base = wid * NCHUNK

    pltpu.sync_copy(urow_hbm.at[pl.ds(base, NCHUNK)], urow_v)
    pltpu.sync_copy(ucol_hbm.at[pl.ds(base, NCHUNK)], ucol_v)
    pltpu.sync_copy(irow_hbm.at[pl.ds(base, NCHUNK)], irow_v)
    pltpu.sync_copy(icol_hbm.at[pl.ds(base, NCHUNK)], icol_v)
    pltpu.sync_copy(b_hbm, b_v)

    bias = b_v[pl.ds(0, L)]
    rows0 = lax.iota(jnp.int32, L)

    for j in range(NCHUNK):
        cu = pltpu.async_copy(us_hbm.at[urow_v.at[j]], usc_v, usem)
        ci = pltpu.async_copy(is_hbm.at[irow_v.at[j]], isc_v, isem)
        cu.wait()
        ci.wait()

        def group_body(g, _, j=j):
            rows = rows0 + g * L
            uv = plsc.load_gather(usc_v, [rows, ucol_v[j, pl.ds(g * L, L)]])
            iv = plsc.load_gather(isc_v, [rows, icol_v[j, pl.ds(g * L, L)]])
            x = uv + iv + bias
            logit_v[pl.ds(j * CHUNK + g * L, L)] = 1.0 / (1.0 + jnp.exp(-x))
            return 0

        lax.fori_loop(0, CHUNK // L, group_body, 0)

    pltpu.sync_copy(logit_v, out_hbm.at[pl.ds(wid * B_PER_W, B_PER_W)])


@jax.jit
def kernel(batch_user_id, batch_item_id, user_table, item_table, W, b):
    uid = batch_user_id.astype(jnp.int32)
    iid = batch_item_id.astype(jnp.int32)
    urow = (uid >> 7).reshape(BATCH // IDXW, IDXW)
    ucol = (uid & 127).reshape(BATCH // IDXW, IDXW)
    irow = (iid >> 7).reshape(BATCH // IDXW, IDXW)
    icol = (iid & 127).reshape(BATCH // IDXW, IDXW)
    b16 = jnp.broadcast_to(b, (L,))

    us2 = _scores(user_table, W[:D].reshape(1, D))
    is2 = _scores(item_table, W[D:].reshape(1, D))

    run = functools.partial(
        pl.kernel,
        out_type=jax.ShapeDtypeStruct((BATCH,), jnp.float32),
        mesh=plsc.VectorSubcoreMesh(core_axis_name="c", subcore_axis_name="s"),
        compiler_params=pltpu.CompilerParams(needs_layout_passes=False),
        scratch_types=[
            pltpu.VMEM((NCHUNK, IDXW), jnp.int32),
            pltpu.VMEM((NCHUNK, IDXW), jnp.int32),
            pltpu.VMEM((NCHUNK, IDXW), jnp.int32),
            pltpu.VMEM((NCHUNK, IDXW), jnp.int32),
            pltpu.VMEM((CHUNK, CHUNK), jnp.float32),
            pltpu.VMEM((CHUNK, CHUNK), jnp.float32),
            pltpu.VMEM((L,), jnp.float32),
            pltpu.VMEM((B_PER_W,), jnp.float32),
            pltpu.SemaphoreType.DMA,
            pltpu.SemaphoreType.DMA,
        ],
    )(_lookup_kernel)
    out = run(urow, ucol, irow, icol, us2, is2, b16)
    return out.reshape(BATCH, 1)


# speedup vs baseline: 1.4094x; 1.3530x over previous
import functools

import jax
import jax.numpy as jnp
from jax import lax
from jax.experimental import pallas as pl
from jax.experimental.pallas import tpu as pltpu
from jax.experimental.pallas import tpu_sc as plsc

BATCH = 16384
NC, NS, L = 2, 16, 16
NW = NC * NS
B_PER_W = BATCH // NW
NG = B_PER_W // L
CHUNK = 128
NCHUNK = B_PER_W // CHUNK
D = 64
IDXW = 128
NSEM = 8
BLKR = 32768


def _score_kernel(w_ref, x_ref, o_ref):
    o_ref[...] = lax.dot_general(
        w_ref[...], x_ref[...],
        dimension_numbers=(((1,), (1,)), ((), ())),
        preferred_element_type=jnp.float32)[None]


def _scores(table, w_half):
    rows = table.shape[0]
    grid = (rows + BLKR - 1) // BLKR
    out = pl.pallas_call(
        _score_kernel,
        grid=(grid,),
        in_specs=[
            pl.BlockSpec((1, D), lambda i: (0, 0)),
            pl.BlockSpec((BLKR, D), lambda i: (i, 0)),
        ],
        out_specs=pl.BlockSpec((1, 1, BLKR), lambda i: (i, 0, 0)),
        out_shape=jax.ShapeDtypeStruct((grid, 1, BLKR), jnp.float32),
    )(w_half, table)
    return out.reshape(grid * BLKR // CHUNK, CHUNK)


def _lr_kernel(uid_hbm, irow_hbm, icol_hbm, utab_hbm, is_hbm, w_hbm, b_hbm,
               out_hbm, uidx_v, irow_v, icol_v, urows_v, isc_v, w_v, b_v,
               logit_v, *sems):
    isem = sems[NSEM]
    wid = lax.axis_index("s") * NC + lax.axis_index("c")
    base = wid * NCHUNK

    pltpu.sync_copy(uid_hbm.at[pl.ds(base, NCHUNK)], uidx_v)
    pltpu.sync_copy(irow_hbm.at[pl.ds(base, NCHUNK)], irow_v)
    pltpu.sync_copy(icol_hbm.at[pl.ds(base, NCHUNK)], icol_v)
    pltpu.sync_copy(w_hbm, w_v)
    pltpu.sync_copy(b_hbm, b_v)

    wu = [w_v[pl.ds(k * L, L)] for k in range(D // L)]

    bias = b_v[pl.ds(0, L)]
    lane = lax.iota(jnp.int32, L)
    perms = [(lane ^ k)[:, None] for k in (8, 4, 2, 1)]
    dnums = lax.GatherDimensionNumbers(
        offset_dims=(), collapsed_slice_dims=(0,), start_index_map=(0,))

    def shuffle(x, p):
        return lax.gather(x, p, dnums, slice_sizes=(1,),
                          mode=lax.GatherScatterMode.PROMISE_IN_BOUNDS)

    def hsum(x):
        for p in perms:
            x = x + shuffle(x, p)
        return x

    NGH = NG // 2
    RH = NGH * L

    for p in range(2):
        def fire_body(g, _, p=p):
            r0 = g * L
            b0 = r0 - p * RH
            mus = uidx_v[r0 // IDXW, pl.ds(r0 % IDXW, L)]
            for l in range(L):
                pltpu.async_copy(
                    utab_hbm.at[pl.ds(mus[l], 1)],
                    urows_v.at[pl.ds(b0 + l, 1)], sems[l % NSEM])
            return 0

        lax.fori_loop(p * NGH, (p + 1) * NGH, fire_body, 0)

        item_copies = []
        for jj in range(RH // CHUNK):
            j = p * (RH // CHUNK) + jj
            item_copies.append(pltpu.async_copy(
                is_hbm.at[irow_v.at[j]],
                isc_v.at[pl.ds(jj * CHUNK, CHUNK)], isem))
        for cp in item_copies:
            cp.wait()

        def group_body(g, _, p=p):
            r0 = g * L
            b0 = r0 - p * RH
            mus = uidx_v[r0 // IDXW, pl.ds(r0 % IDXW, L)]
            mic = icol_v[r0 // IDXW, pl.ds(r0 % IDXW, L)]
            for l in range(L):
                pltpu.make_async_copy(
                    utab_hbm.at[pl.ds(mus[l], 1)],
                    urows_v.at[pl.ds(b0 + l, 1)], sems[l % NSEM]).wait()
            vec = bias
            for l in range(L):
                acc = urows_v[b0 + l, pl.ds(0, L)] * wu[0]
                for k in range(1, D // L):
                    acc += urows_v[b0 + l, pl.ds(k * L, L)] * wu[k]
                c = mic[l]
                cv = isc_v[b0 + l, pl.ds(c & 112, L)]
                ivb = shuffle(cv, jnp.full((L, 1), c & 15, jnp.int32))
                vec += jnp.where(lane == l, hsum(acc) + ivb, 0.0)
            logit_v[pl.ds(r0, L)] = 1.0 / (1.0 + jnp.exp(-vec))
            return 0

        lax.fori_loop(p * NGH, (p + 1) * NGH, group_body, 0)

    pltpu.sync_copy(logit_v, out_hbm.at[pl.ds(wid * B_PER_W, B_PER_W)])


@jax.jit
def kernel(batch_user_id, batch_item_id, user_table, item_table, W, b):
    uid = batch_user_id.astype(jnp.int32)
    iid = batch_item_id.astype(jnp.int32)
    uid2 = uid.reshape(BATCH // IDXW, IDXW)
    irow = (iid >> 7).reshape(BATCH // IDXW, IDXW)
    icol = (iid & 127).reshape(BATCH // IDXW, IDXW)
    w = W.reshape(2 * D)
    b16 = jnp.broadcast_to(b, (L,))

    is2 = _scores(item_table, W[D:].reshape(1, D))

    out = functools.partial(
        pl.kernel,
        out_type=jax.ShapeDtypeStruct((BATCH,), jnp.float32),
        mesh=plsc.VectorSubcoreMesh(core_axis_name="c", subcore_axis_name="s"),
        scratch_types=[
            pltpu.VMEM((NCHUNK, IDXW), jnp.int32),
            pltpu.VMEM((NCHUNK, IDXW), jnp.int32),
            pltpu.VMEM((NCHUNK, IDXW), jnp.int32),
            pltpu.VMEM((B_PER_W // 2, D), jnp.float32),
            pltpu.VMEM((B_PER_W // 2, CHUNK), jnp.float32),
            pltpu.VMEM((2 * D,), jnp.float32),
            pltpu.VMEM((L,), jnp.float32),
            pltpu.VMEM((B_PER_W,), jnp.float32),
        ] + [pltpu.SemaphoreType.DMA] * (NSEM + 1),
    )(_lr_kernel)(uid2, irow, icol, user_table, is2, w, b16)
    return out.reshape(BATCH, 1)


# speedup vs baseline: 1.5047x; 1.0677x over previous
import functools

import jax
import jax.numpy as jnp
from jax import lax
from jax.experimental import pallas as pl
from jax.experimental.pallas import tpu as pltpu
from jax.experimental.pallas import tpu_sc as plsc

BATCH = 16384
NC, NS, L = 2, 16, 16
NW = NC * NS
B_PER_W = BATCH // NW
NG = B_PER_W // L
D = 64
IDXW = 128
NSEM = 8


def _lr_kernel(uid_hbm, iid_hbm, utab_hbm, itab_hbm, w_hbm, b_hbm, out_hbm,
               uidx_v, iidx_v, urows_v, irows_v, w_v, b_v, logit_v,
               *sems):
    wid = lax.axis_index("s") * NC + lax.axis_index("c")
    base = wid * (B_PER_W // IDXW)

    pltpu.sync_copy(uid_hbm.at[pl.ds(base, B_PER_W // IDXW)], uidx_v)
    pltpu.sync_copy(iid_hbm.at[pl.ds(base, B_PER_W // IDXW)], iidx_v)
    pltpu.sync_copy(w_hbm, w_v)
    pltpu.sync_copy(b_hbm, b_v)

    wu = [w_v[pl.ds(k * L, L)] for k in range(D // L)]
    wi = [w_v[pl.ds(D + k * L, L)] for k in range(D // L)]

    bias = b_v[pl.ds(0, L)]
    lane = lax.iota(jnp.int32, L)
    perms = [(lane ^ k)[:, None] for k in (8, 4, 2, 1)]
    dnums = lax.GatherDimensionNumbers(
        offset_dims=(), collapsed_slice_dims=(0,), start_index_map=(0,))

    def hsum(x):
        for p in perms:
            x = x + lax.gather(x, p, dnums, slice_sizes=(1,),
                               mode=lax.GatherScatterMode.PROMISE_IN_BOUNDS)
        return x

    NGH = NG // 2
    RH = NGH * L

    for p in range(2):
        def fire_body(g, _, p=p):
            r0 = g * L
            b0 = r0 - p * RH
            mus = uidx_v[r0 // IDXW, pl.ds(r0 % IDXW, L)]
            mis = iidx_v[r0 // IDXW, pl.ds(r0 % IDXW, L)]
            for l in range(L):
                pltpu.async_copy(
                    utab_hbm.at[pl.ds(mus[l], 1)],
                    urows_v.at[pl.ds(b0 + l, 1)], sems[l % (NSEM // 2)])
                pltpu.async_copy(
                    itab_hbm.at[pl.ds(mis[l], 1)],
                    irows_v.at[pl.ds(b0 + l, 1)],
                    sems[NSEM // 2 + l % (NSEM // 2)])
            return 0

        lax.fori_loop(p * NGH, (p + 1) * NGH, fire_body, 0)

        def group_body(g, _, p=p):
            r0 = g * L
            b0 = r0 - p * RH
            mus = uidx_v[r0 // IDXW, pl.ds(r0 % IDXW, L)]
            mis = iidx_v[r0 // IDXW, pl.ds(r0 % IDXW, L)]
            for l in range(L):
                pltpu.make_async_copy(
                    utab_hbm.at[pl.ds(mus[l], 1)],
                    urows_v.at[pl.ds(b0 + l, 1)],
                    sems[l % (NSEM // 2)]).wait()
                pltpu.make_async_copy(
                    itab_hbm.at[pl.ds(mis[l], 1)],
                    irows_v.at[pl.ds(b0 + l, 1)],
                    sems[NSEM // 2 + l % (NSEM // 2)]).wait()
            vec = bias
            for l in range(L):
                acc = urows_v[b0 + l, pl.ds(0, L)] * wu[0]
                for k in range(1, D // L):
                    acc += urows_v[b0 + l, pl.ds(k * L, L)] * wu[k]
                for k in range(D // L):
                    acc += irows_v[b0 + l, pl.ds(k * L, L)] * wi[k]
                vec += jnp.where(lane == l, hsum(acc), 0.0)
            logit_v[pl.ds(r0, L)] = 1.0 / (1.0 + jnp.exp(-vec))
            return 0

        lax.fori_loop(p * NGH, (p + 1) * NGH, group_body, 0)

    pltpu.sync_copy(logit_v, out_hbm.at[pl.ds(wid * B_PER_W, B_PER_W)])


@jax.jit
def kernel(batch_user_id, batch_item_id, user_table, item_table, W, b):
    uid2 = batch_user_id.astype(jnp.int32).reshape(BATCH // IDXW, IDXW)
    iid2 = batch_item_id.astype(jnp.int32).reshape(BATCH // IDXW, IDXW)
    w = W.reshape(2 * D)
    b16 = jnp.broadcast_to(b, (L,))

    run = functools.partial(
        pl.kernel,
        out_type=jax.ShapeDtypeStruct((BATCH,), jnp.float32),
        mesh=plsc.VectorSubcoreMesh(core_axis_name="c", subcore_axis_name="s"),
        scratch_types=[
            pltpu.VMEM((B_PER_W // IDXW, IDXW), jnp.int32),
            pltpu.VMEM((B_PER_W // IDXW, IDXW), jnp.int32),
            pltpu.VMEM((B_PER_W // 2, D), jnp.float32),
            pltpu.VMEM((B_PER_W // 2, D), jnp.float32),
            pltpu.VMEM((2 * D,), jnp.float32),
            pltpu.VMEM((L,), jnp.float32),
            pltpu.VMEM((B_PER_W,), jnp.float32),
        ] + [pltpu.SemaphoreType.DMA] * NSEM,
    )(_lr_kernel)
    out = run(uid2, iid2, user_table, item_table, w, b16)
    return out.reshape(BATCH, 1)
